# baseline (speedup 1.0000x reference)
import functools

import jax
import jax.numpy as jnp
from jax import lax
from jax.experimental import pallas as pl
from jax.experimental.pallas import tpu as pltpu
from jax.experimental.pallas import tpu_sc as plsc

_N = 10000
_E = 320000
_D = 128
_NC = 2
_NS = 16
_NW = _NC * _NS
_EPW = _E // _NW
_K = 80
_NCHUNK = _EPW // _K
_NPAD = 10240
_RPS = _NPAD // _NS

_PREC = lax.Precision.DEFAULT


def _relu(v):
    return jnp.maximum(v, 0.0)


def _dot(a, b):
    return jnp.dot(a, b, preferred_element_type=jnp.float32, precision=_PREC)


def _sc_segment_sum(g, src3, dst3, zeros):
    mesh = plsc.VectorSubcoreMesh(core_axis_name="c", subcore_axis_name="s")

    @functools.partial(
        pl.kernel,
        out_type=jax.ShapeDtypeStruct((_NC, _NPAD, _D), jnp.float32),
        mesh=mesh,
        scratch_types=[
            pltpu.VMEM_SHARED((_NPAD, _D), jnp.float32),
            pltpu.VMEM((8, _K), jnp.int32),
            pltpu.VMEM((8, _K), jnp.int32),
            pltpu.VMEM((4, _K, _D), jnp.float32),
            pltpu.SemaphoreType.DMA,
            pltpu.SemaphoreType.DMA,
            pltpu.SemaphoreType.DMA,
        ],
    )
    def sc_kernel(g_hbm, src_hbm, dst_hbm, z_hbm, out_hbm, acc, si, di, rows,
                  gsem, ssem, isem):
        c = lax.axis_index("c")
        s = lax.axis_index("s")
        wid = s * _NC + c
        base = wid * _EPW

        def idx_slices(j):
            return (src_hbm.at[pl.ds(base + j * _K, _K)],
                    dst_hbm.at[pl.ds(base + j * _K, _K)])

        def start_idx(j, slot):
            sh, dh = idx_slices(j)
            pltpu.make_async_copy(sh, si.at[slot], isem).start()
            pltpu.make_async_copy(dh, di.at[slot], isem).start()

        def drain_idx(j, slot):
            sh, dh = idx_slices(j)
            pltpu.make_async_copy(sh, si.at[slot], isem).wait()
            pltpu.make_async_copy(dh, di.at[slot], isem).wait()

        def sync_idx(j, slot):
            sh, dh = idx_slices(j)
            pltpu.sync_copy(sh, si.at[slot])
            pltpu.sync_copy(dh, di.at[slot])

        def start_gather(j, buf, slot):
            pltpu.make_async_copy(g_hbm.at[si.at[slot]], rows.at[buf],
                                  gsem).start()

        def drain_gather(j, buf, slot):
            pltpu.make_async_copy(g_hbm.at[si.at[slot]], rows.at[buf],
                                  gsem).wait()

        def scatter_chunk(j, buf, slot):
            pltpu.make_async_copy(rows.at[buf], acc.at[di.at[slot]],
                                  ssem).start(add=True)
            pltpu.make_async_copy(rows.at[buf], acc.at[di.at[slot]],
                                  ssem).wait()

        for m in range(3):
            sync_idx(m, m)
        start_idx(3, 3)
        start_idx(4, 4)
        for m in range(3):
            start_gather(m, m, m)
        pltpu.sync_copy(z_hbm, acc.at[pl.ds(s * _RPS, _RPS)])
        plsc.subcore_barrier()

        @pl.loop(0, _NCHUNK - 5, step=8)
        def _(j0):
            for k in range(8):
                j = j0 + k
                drain_gather(j, k % 4, k % 8)
                start_idx(j + 5, (k + 5) % 8)
                drain_idx(j + 3, (k + 3) % 8)
                start_gather(j + 3, (k + 3) % 4, (k + 3) % 8)
                scatter_chunk(j, k % 4, k % 8)

        drain_gather(_NCHUNK - 5, 0, 0)
        drain_idx(_NCHUNK - 2, 3)
        start_gather(_NCHUNK - 2, 3, 3)
        scatter_chunk(_NCHUNK - 5, 0, 0)

        drain_gather(_NCHUNK - 4, 1, 1)
        drain_idx(_NCHUNK - 1, 4)
        start_gather(_NCHUNK - 1, 0, 4)
        scatter_chunk(_NCHUNK - 4, 1, 1)

        drain_gather(_NCHUNK - 3, 2, 2)
        scatter_chunk(_NCHUNK - 3, 2, 2)

        drain_gather(_NCHUNK - 2, 3, 3)
        scatter_chunk(_NCHUNK - 2, 3, 3)

        drain_gather(_NCHUNK - 1, 0, 4)
        scatter_chunk(_NCHUNK - 1, 0, 4)

        plsc.subcore_barrier()
        row0 = s * _RPS
        pltpu.sync_copy(acc.at[pl.ds(row0, _RPS)],
                        out_hbm.at[c, pl.ds(row0, _RPS)])

    return sc_kernel(g, src3, dst3, zeros)


def _tc_enc_msg_body(x_ref, we_ref, be_ref, wm_ref, bm_ref, h_ref, g_ref):
    h = _relu(_dot(x_ref[...], we_ref[...]) + be_ref[...])
    h_ref[...] = h
    g_ref[...] = _relu(_dot(h, wm_ref[...]) + bm_ref[...])


_BN = 2000
_GRID = _N // _BN

_row_spec = pl.BlockSpec((_BN, _D), lambda i: (i, 0))
_p_spec = pl.BlockSpec((_NC, _BN, _D), lambda i: (0, i, 0))
_w_spec = pl.BlockSpec((_D, _D), lambda i: (0, 0))
_b_spec = pl.BlockSpec((1, _D), lambda i: (0, 0))


def _tc_enc_msg(x, W_enc, b_enc, Wm, bm):
    return pl.pallas_call(
        _tc_enc_msg_body,
        grid=(_GRID,),
        in_specs=[_row_spec, _w_spec, _b_spec, _w_spec, _b_spec],
        out_specs=[_row_spec, _row_spec],
        out_shape=[
            jax.ShapeDtypeStruct((_N, _D), jnp.float32),
            jax.ShapeDtypeStruct((_N, _D), jnp.float32),
        ],
    )(x, W_enc, b_enc.reshape(1, _D), Wm, bm.reshape(1, _D))


def _tc_upd_msg_body(h_ref, p_ref, wuh_ref, wua_ref, bu_ref, wm_ref, bm_ref,
                     h2_ref, g_ref):
    agg = p_ref[0] + p_ref[1]
    u = _relu(_dot(h_ref[...], wuh_ref[...]) + _dot(agg, wua_ref[...])
              + bu_ref[...])
    h2 = h_ref[...] + u
    h2_ref[...] = h2
    g_ref[...] = _relu(_dot(h2, wm_ref[...]) + bm_ref[...])


def _tc_upd_msg(h, p, Wu, bu, Wm, bm):
    return pl.pallas_call(
        _tc_upd_msg_body,
        grid=(_GRID,),
        in_specs=[_row_spec, _p_spec, _w_spec, _w_spec, _b_spec, _w_spec,
                  _b_spec],
        out_specs=[_row_spec, _row_spec],
        out_shape=[
            jax.ShapeDtypeStruct((_N, _D), jnp.float32),
            jax.ShapeDtypeStruct((_N, _D), jnp.float32),
        ],
    )(h, p, Wu[:_D], Wu[_D:], bu.reshape(1, _D), Wm, bm.reshape(1, _D))


def _tc_upd_head_body(h_ref, p_ref, wuh_ref, wua_ref, bu_ref, wh_ref, bh_ref,
                      o_ref):
    agg = p_ref[0] + p_ref[1]
    u = _relu(_dot(h_ref[...], wuh_ref[...]) + _dot(agg, wua_ref[...])
              + bu_ref[...])
    h2 = h_ref[...] + u
    o_ref[...] = _dot(h2, wh_ref[...]) + bh_ref[...]


def _tc_upd_head(h, p, Wu, bu, W_head, b_head):
    return pl.pallas_call(
        _tc_upd_head_body,
        grid=(_GRID,),
        in_specs=[_row_spec, _p_spec, _w_spec, _w_spec, _b_spec, _w_spec,
                  _b_spec],
        out_specs=_row_spec,
        out_shape=jax.ShapeDtypeStruct((_N, _D), jnp.float32),
    )(h, p, Wu[:_D], Wu[_D:], bu.reshape(1, _D), W_head, b_head.reshape(1, _D))


def kernel(x, edge_index, W_enc, b_enc, W_msg0, b_msg0, W_upd0, b_upd0,
           W_msg1, b_msg1, W_upd1, b_upd1, W_msg2, b_msg2, W_upd2, b_upd2,
           W_head, b_head):
    src = edge_index[0]
    dst = edge_index[1]
    zeros = jnp.zeros((_RPS, _D), dtype=jnp.float32)

    h, g = _tc_enc_msg(x, W_enc, b_enc, W_msg0, b_msg0)
    p = _sc_segment_sum(g, src, dst, zeros)
    h, g = _tc_upd_msg(h, p, W_upd0, b_upd0, W_msg1, b_msg1)
    p = _sc_segment_sum(g, src, dst, zeros)
    h, g = _tc_upd_msg(h, p, W_upd1, b_upd1, W_msg2, b_msg2)
    p = _sc_segment_sum(g, src, dst, zeros)
    return _tc_upd_head(h, p, W_upd2, b_upd2, W_head, b_head)


# speedup vs baseline: 1.0030x; 1.0030x over previous
import functools

import jax
import jax.numpy as jnp
from jax import lax
from jax.experimental import pallas as pl
from jax.experimental.pallas import tpu as pltpu
from jax.experimental.pallas import tpu_sc as plsc

_N = 10000
_E = 320000
_D = 128
_NC = 2
_NS = 16
_NW = _NC * _NS
_EPW = _E // _NW
_K = 80
_NCHUNK = _EPW // _K
_NPAD = 10240
_RPS = _NPAD // _NS

_PREC = lax.Precision.DEFAULT


def _relu(v):
    return jnp.maximum(v, 0.0)


def _dot(a, b):
    return jnp.dot(a, b, preferred_element_type=jnp.float32, precision=_PREC)


def _sc_segment_sum(g, src3, dst3, zeros):
    mesh = plsc.VectorSubcoreMesh(core_axis_name="c", subcore_axis_name="s")

    @functools.partial(
        pl.kernel,
        out_type=jax.ShapeDtypeStruct((_NC, _NPAD, _D), jnp.float32),
        mesh=mesh,
        scratch_types=[
            pltpu.VMEM_SHARED((_NPAD, _D), jnp.float32),
            pltpu.VMEM((8, _K), jnp.int32),
            pltpu.VMEM((8, _K), jnp.int32),
            pltpu.VMEM((4, _K, _D), jnp.float32),
            pltpu.SemaphoreType.DMA,
            pltpu.SemaphoreType.DMA,
            pltpu.SemaphoreType.DMA,
        ],
    )
    def sc_kernel(g_hbm, src_hbm, dst_hbm, z_hbm, out_hbm, acc, si, di, rows,
                  gsem, ssem, isem):
        c = lax.axis_index("c")
        s = lax.axis_index("s")
        wid = s * _NC + c
        base = wid * _EPW

        def idx_slices(j):
            return (src_hbm.at[pl.ds(base + j * _K, _K)],
                    dst_hbm.at[pl.ds(base + j * _K, _K)])

        def start_idx(j, slot):
            sh, dh = idx_slices(j)
            pltpu.make_async_copy(sh, si.at[slot], isem).start()
            pltpu.make_async_copy(dh, di.at[slot], isem).start()

        def drain_idx(j, slot):
            sh, dh = idx_slices(j)
            pltpu.make_async_copy(sh, si.at[slot], isem).wait()
            pltpu.make_async_copy(dh, di.at[slot], isem).wait()

        def sync_idx(j, slot):
            sh, dh = idx_slices(j)
            pltpu.sync_copy(sh, si.at[slot])
            pltpu.sync_copy(dh, di.at[slot])

        def start_gather(j, buf, slot):
            pltpu.make_async_copy(g_hbm.at[si.at[slot]], rows.at[buf],
                                  gsem).start()

        def drain_gather(j, buf, slot):
            pltpu.make_async_copy(g_hbm.at[si.at[slot]], rows.at[buf],
                                  gsem).wait()

        def scatter_chunk(j, buf, slot):
            pltpu.make_async_copy(rows.at[buf], acc.at[di.at[slot]],
                                  ssem).start(add=True)
            pltpu.make_async_copy(rows.at[buf], acc.at[di.at[slot]],
                                  ssem).wait()

        for m in range(3):
            sync_idx(m, m)
        start_idx(3, 3)
        start_idx(4, 4)
        for m in range(3):
            start_gather(m, m, m)
        pltpu.sync_copy(z_hbm, acc.at[pl.ds(s * _RPS, _RPS)])
        plsc.subcore_barrier()

        @pl.loop(0, _NCHUNK - 5, step=8)
        def _(j0):
            for k in range(8):
                j = j0 + k
                drain_gather(j, k % 4, k % 8)
                start_idx(j + 5, (k + 5) % 8)
                drain_idx(j + 3, (k + 3) % 8)
                start_gather(j + 3, (k + 3) % 4, (k + 3) % 8)
                scatter_chunk(j, k % 4, k % 8)

        drain_gather(_NCHUNK - 5, 0, 0)
        drain_idx(_NCHUNK - 2, 3)
        start_gather(_NCHUNK - 2, 3, 3)
        scatter_chunk(_NCHUNK - 5, 0, 0)

        drain_gather(_NCHUNK - 4, 1, 1)
        drain_idx(_NCHUNK - 1, 4)
        start_gather(_NCHUNK - 1, 0, 4)
        scatter_chunk(_NCHUNK - 4, 1, 1)

        drain_gather(_NCHUNK - 3, 2, 2)
        scatter_chunk(_NCHUNK - 3, 2, 2)

        drain_gather(_NCHUNK - 2, 3, 3)
        scatter_chunk(_NCHUNK - 2, 3, 3)

        drain_gather(_NCHUNK - 1, 0, 4)
        scatter_chunk(_NCHUNK - 1, 0, 4)

        plsc.subcore_barrier()
        row0 = s * _RPS
        pltpu.sync_copy(acc.at[pl.ds(row0, _RPS)],
                        out_hbm.at[c, pl.ds(row0, _RPS)])

    return sc_kernel(g, src3, dst3, zeros)


def _tc_enc_msg_body(x_ref, we_ref, be_ref, wm_ref, bm_ref, h_ref, g_ref):
    h = _relu(_dot(x_ref[...], we_ref[...]) + be_ref[...])
    h_ref[...] = h
    g_ref[...] = _relu(_dot(h, wm_ref[...]) + bm_ref[...])


def _tc_enc_msg(x, W_enc, b_enc, Wm, bm):
    return pl.pallas_call(
        _tc_enc_msg_body,
        out_shape=[
            jax.ShapeDtypeStruct((_N, _D), jnp.float32),
            jax.ShapeDtypeStruct((_N, _D), jnp.float32),
        ],
    )(x, W_enc, b_enc.reshape(1, _D), Wm, bm.reshape(1, _D))


def _tc_upd_msg_body(h_ref, p_ref, wuh_ref, wua_ref, bu_ref, wm_ref, bm_ref,
                     h2_ref, g_ref):
    agg = p_ref[0, :_N] + p_ref[1, :_N]
    u = _relu(_dot(h_ref[...], wuh_ref[...]) + _dot(agg, wua_ref[...])
              + bu_ref[...])
    h2 = h_ref[...] + u
    h2_ref[...] = h2
    g_ref[...] = _relu(_dot(h2, wm_ref[...]) + bm_ref[...])


def _tc_upd_msg(h, p, Wu, bu, Wm, bm):
    return pl.pallas_call(
        _tc_upd_msg_body,
        out_shape=[
            jax.ShapeDtypeStruct((_N, _D), jnp.float32),
            jax.ShapeDtypeStruct((_N, _D), jnp.float32),
        ],
    )(h, p, Wu[:_D], Wu[_D:], bu.reshape(1, _D), Wm, bm.reshape(1, _D))


def _tc_upd_head_body(h_ref, p_ref, wuh_ref, wua_ref, bu_ref, wh_ref, bh_ref,
                      o_ref):
    agg = p_ref[0, :_N] + p_ref[1, :_N]
    u = _relu(_dot(h_ref[...], wuh_ref[...]) + _dot(agg, wua_ref[...])
              + bu_ref[...])
    h2 = h_ref[...] + u
    o_ref[...] = _dot(h2, wh_ref[...]) + bh_ref[...]


def _tc_upd_head(h, p, Wu, bu, W_head, b_head):
    return pl.pallas_call(
        _tc_upd_head_body,
        out_shape=jax.ShapeDtypeStruct((_N, _D), jnp.float32),
    )(h, p, Wu[:_D], Wu[_D:], bu.reshape(1, _D), W_head, b_head.reshape(1, _D))


def kernel(x, edge_index, W_enc, b_enc, W_msg0, b_msg0, W_upd0, b_upd0,
           W_msg1, b_msg1, W_upd1, b_upd1, W_msg2, b_msg2, W_upd2, b_upd2,
           W_head, b_head):
    src = edge_index[0]
    dst = edge_index[1]
    zeros = jnp.zeros((_RPS, _D), dtype=jnp.float32)

    h, g = _tc_enc_msg(x, W_enc, b_enc, W_msg0, b_msg0)
    p = _sc_segment_sum(g, src, dst, zeros)
    h, g = _tc_upd_msg(h, p, W_upd0, b_upd0, W_msg1, b_msg1)
    p = _sc_segment_sum(g, src, dst, zeros)
    h, g = _tc_upd_msg(h, p, W_upd1, b_upd1, W_msg2, b_msg2)
    p = _sc_segment_sum(g, src, dst, zeros)
    return _tc_upd_head(h, p, W_upd2, b_upd2, W_head, b_head)
